# Initial kernel scaffold; baseline (speedup 1.0000x reference)
import jax
import jax.numpy as jnp
from jax.experimental import pallas as pl


def kernel(x, edge_index, edge_attr, node_attr, W_in, b_in, W_out, b_out):
    raise NotImplementedError("write your pallas kernel here")


# speedup vs baseline: 1.5375x; 1.5375x over previous
import functools

import jax
import jax.numpy as jnp
from jax import lax
from jax.experimental import pallas as pl
from jax.experimental.pallas import tpu as pltpu
from jax.experimental.pallas import tpu_sc as plsc

_LANES = 16
_NC = 2
_NS = 16
_CHUNK = 80


def _tc_scaling_body(ea_ref, w_ref, b_ref, out_ref, *, icp):
    s = jnp.dot(ea_ref[...], w_ref[...], preferred_element_type=jnp.float32)
    s = jnp.maximum(s + b_ref[...], 0.0)
    out_ref[0] = s[:, :icp]
    out_ref[1] = s[:, icp:]


def _tc_out_body(a0_ref, a1_ref, w0_ref, w1_ref, b_ref, out_ref):
    acc = jnp.dot(a0_ref[...], w0_ref[...], preferred_element_type=jnp.float32)
    acc = acc + jnp.dot(a1_ref[...], w1_ref[...], preferred_element_type=jnp.float32)
    out_ref[...] = jnp.tanh(acc + b_ref[...])


def _sc_gather_scale_scatter(e, icp, npad):
    ept = e // _NS
    nchunk = ept // _CHUNK
    rpt = npad // _NS
    nvec = icp // _LANES
    zrows = _CHUNK

    mesh = plsc.VectorSubcoreMesh(
        core_axis_name="c", subcore_axis_name="s",
        num_cores=_NC, num_subcores=_NS)

    @functools.partial(
        pl.kernel,
        out_type=jax.ShapeDtypeStruct((_NC * npad, icp), jnp.float32),
        mesh=mesh,
        scratch_types=[
            pltpu.VMEM((_CHUNK,), jnp.int32),
            pltpu.VMEM((_CHUNK,), jnp.int32),
            pltpu.VMEM((_CHUNK, icp), jnp.float32),
            pltpu.VMEM((_CHUNK, icp), jnp.float32),
            pltpu.VMEM_SHARED((npad, icp), jnp.float32),
            pltpu.VMEM((zrows, icp), jnp.float32),
            pltpu.SemaphoreType.DMA,
        ],
        compiler_params=pltpu.CompilerParams(use_tc_tiling_on_sc=False),
    )
    def sc_kernel(xc_hbm, src_hbm, dst_hbm, s_hbm, out_hbm,
                  srcv, dstv, sv, xv, acc, zbuf, sem):
        c = lax.axis_index("c")
        s = lax.axis_index("s")

        def zrow(r, _):
            for k in range(nvec):
                zbuf[r, pl.ds(k * _LANES, _LANES)] = jnp.zeros(
                    (_LANES,), jnp.float32)
            return 0
        lax.fori_loop(0, zrows, zrow, 0)

        def zcopy(j, _):
            pltpu.sync_copy(
                zbuf, acc.at[pl.ds(s * rpt + j * zrows, zrows), :])
            return 0
        lax.fori_loop(0, rpt // zrows, zcopy, 0)
        plsc.subcore_barrier()

        def chunk(j, _):
            base = s * ept + j * _CHUNK
            pltpu.sync_copy(src_hbm.at[pl.ds(base, _CHUNK)], srcv)
            pltpu.sync_copy(dst_hbm.at[pl.ds(base, _CHUNK)], dstv)
            pltpu.sync_copy(s_hbm.at[pl.ds(c * e + base, _CHUNK), :], sv)
            pltpu.async_copy(xc_hbm.at[srcv], xv, sem).wait()

            def erow(i, _):
                for k in range(nvec):
                    sl = pl.ds(k * _LANES, _LANES)
                    sv[i, sl] = sv[i, sl] * xv[i, sl]
                return 0
            lax.fori_loop(0, _CHUNK, erow, 0)

            pltpu.sync_copy(sv, acc.at[dstv], add=True)
            return 0
        lax.fori_loop(0, nchunk, chunk, 0)
        plsc.subcore_barrier()

        pltpu.sync_copy(
            acc.at[pl.ds(s * rpt, rpt), :],
            out_hbm.at[pl.ds(c * npad + s * rpt, rpt), :])

    return sc_kernel


def kernel(x, edge_index, edge_attr, node_attr, W_in, b_in, W_out, b_out):
    n, d = x.shape
    na = node_attr.shape[1]
    e = edge_index.shape[1]
    ea = edge_attr.shape[1]
    ic = d + na
    oc = W_out.shape[0]
    icp = ((ic + _LANES - 1) // _LANES) * _LANES
    npad = ((n + _NS * 8 - 1) // (_NS * 8)) * (_NS * 8)
    npad = ((npad + _NS * _CHUNK - 1) // (_NS * _CHUNK)) * (_NS * _CHUNK)

    f32 = jnp.float32
    src = edge_index[0].astype(jnp.int32)
    dst = edge_index[1].astype(jnp.int32)

    xc = jnp.concatenate([x.astype(f32), node_attr.astype(f32)], axis=1)
    xcp = jnp.pad(xc, ((0, 0), (0, icp - ic)))

    w_h = [jnp.pad(W_in[h::2, :], ((0, icp - ic), (0, 0))) for h in range(2)]
    w_cat = jnp.concatenate(w_h, axis=0).T.astype(f32)
    b_h = [jnp.pad(b_in[h::2], (0, icp - ic)) for h in range(2)]
    b_cat = jnp.concatenate(b_h, axis=0)[None, :].astype(f32)

    tile_e = 640
    scal = pl.pallas_call(
        functools.partial(_tc_scaling_body, icp=icp),
        grid=(e // tile_e,),
        in_specs=[
            pl.BlockSpec((tile_e, ea), lambda i: (i, 0)),
            pl.BlockSpec((ea, 2 * icp), lambda i: (0, 0)),
            pl.BlockSpec((1, 2 * icp), lambda i: (0, 0)),
        ],
        out_specs=pl.BlockSpec((2, tile_e, icp), lambda i: (0, i, 0)),
        out_shape=jax.ShapeDtypeStruct((2, e, icp), f32),
    )(edge_attr.astype(f32), w_cat, b_cat)
    scal_flat = scal.reshape(2 * e, icp)

    sc_fn = _sc_gather_scale_scatter(e, icp, npad)
    aggr = sc_fn(xcp, src, dst, scal_flat)
    a0 = aggr[:n]
    a1 = aggr[npad:npad + n]

    w0o = jnp.pad(W_out[:, 0::2], ((0, 0), (0, icp - ic))).T.astype(f32)
    w1o = jnp.pad(W_out[:, 1::2], ((0, 0), (0, icp - ic))).T.astype(f32)
    b_o = b_out[None, :].astype(f32)

    tile_n = 1000
    out = pl.pallas_call(
        _tc_out_body,
        grid=(n // tile_n,),
        in_specs=[
            pl.BlockSpec((tile_n, icp), lambda i: (i, 0)),
            pl.BlockSpec((tile_n, icp), lambda i: (i, 0)),
            pl.BlockSpec((icp, oc), lambda i: (0, 0)),
            pl.BlockSpec((icp, oc), lambda i: (0, 0)),
            pl.BlockSpec((1, oc), lambda i: (0, 0)),
        ],
        out_specs=pl.BlockSpec((tile_n, oc), lambda i: (i, 0)),
        out_shape=jax.ShapeDtypeStruct((n, oc), f32),
    )(a0, a1, w0o, w1o, b_o)
    return out


# speedup vs baseline: 1.7219x; 1.1200x over previous
import functools

import jax
import jax.numpy as jnp
from jax import lax
from jax.experimental import pallas as pl
from jax.experimental.pallas import tpu as pltpu
from jax.experimental.pallas import tpu_sc as plsc

_LANES = 16
_NC = 2
_NS = 16
_CHUNK = 40


def _tc_scaling_body(ea_ref, w_ref, b_ref, out_ref, *, icp):
    s = jnp.dot(ea_ref[...], w_ref[...], preferred_element_type=jnp.float32)
    s = jnp.maximum(s + b_ref[...], 0.0)
    out_ref[0] = s[:, :icp]
    out_ref[1] = s[:, icp:]


def _tc_out_body(a0_ref, a1_ref, w0_ref, w1_ref, b_ref, out_ref):
    acc = jnp.dot(a0_ref[...], w0_ref[...], preferred_element_type=jnp.float32)
    acc = acc + jnp.dot(a1_ref[...], w1_ref[...], preferred_element_type=jnp.float32)
    out_ref[...] = jnp.tanh(acc + b_ref[...])


def _sc_gather_scale_scatter(e, icp, npad):
    ept = e // _NS
    nchunk = ept // _CHUNK
    rpt = npad // _NS
    nvec = icp // _LANES
    zrows = 16

    npairs = nchunk // 2

    mesh = plsc.VectorSubcoreMesh(
        core_axis_name="c", subcore_axis_name="s",
        num_cores=_NC, num_subcores=_NS)

    @functools.partial(
        pl.kernel,
        out_type=jax.ShapeDtypeStruct((_NC * npad, icp), jnp.float32),
        mesh=mesh,
        scratch_types=[
            pltpu.VMEM((_CHUNK,), jnp.int32),
            pltpu.VMEM((_CHUNK,), jnp.int32),
            pltpu.VMEM((_CHUNK,), jnp.int32),
            pltpu.VMEM((_CHUNK,), jnp.int32),
            pltpu.VMEM((_CHUNK, icp), jnp.float32),
            pltpu.VMEM((_CHUNK, icp), jnp.float32),
            pltpu.VMEM((_CHUNK, icp), jnp.float32),
            pltpu.VMEM((_CHUNK, icp), jnp.float32),
            pltpu.VMEM_SHARED((npad, icp), jnp.float32),
            pltpu.VMEM((zrows, icp), jnp.float32),
            pltpu.SemaphoreType.DMA,
            pltpu.SemaphoreType.DMA,
            pltpu.SemaphoreType.DMA,
            pltpu.SemaphoreType.DMA,
        ],
        compiler_params=pltpu.CompilerParams(use_tc_tiling_on_sc=False),
    )
    def sc_kernel(xc_hbm, src_hbm, dst_hbm, s_hbm, out_hbm,
                  srcv0, srcv1, dstv0, dstv1, sv0, sv1, xv0, xv1,
                  acc, zbuf, ls0, ls1, gs0, gs1):
        c = lax.axis_index("c")
        s = lax.axis_index("s")
        srcv = (srcv0, srcv1)
        dstv = (dstv0, dstv1)
        sv = (sv0, sv1)
        xv = (xv0, xv1)
        ls = (ls0, ls1)
        gs = (gs0, gs1)

        def zrow(r, _):
            for k in range(nvec):
                zbuf[r, pl.ds(k * _LANES, _LANES)] = jnp.zeros(
                    (_LANES,), jnp.float32)
            return 0
        lax.fori_loop(0, zrows, zrow, 0)

        def zcopy(j, _):
            pltpu.sync_copy(
                zbuf, acc.at[pl.ds(s * rpt + j * zrows, zrows), :])
            return 0
        lax.fori_loop(0, rpt // zrows, zcopy, 0)
        plsc.subcore_barrier()

        def loads_descr(cj, b):
            base = s * ept + cj * _CHUNK
            return (
                pltpu.make_async_copy(
                    src_hbm.at[pl.ds(base, _CHUNK)], srcv[b], ls[b]),
                pltpu.make_async_copy(
                    dst_hbm.at[pl.ds(base, _CHUNK)], dstv[b], ls[b]),
                pltpu.make_async_copy(
                    s_hbm.at[pl.ds(c * e + base, _CHUNK), :], sv[b], ls[b]),
            )

        def issue_loads(cj, b):
            for d in loads_descr(cj, b):
                d.start()

        def wait_loads(cj, b):
            for d in loads_descr(cj, b):
                d.wait()

        def gather_descr(b):
            return pltpu.make_async_copy(xc_hbm.at[srcv[b]], xv[b], gs[b])

        def process(cj, b, nb):
            gather_descr(b).wait()

            def erow(i, _):
                for k in range(nvec):
                    sl = pl.ds(k * _LANES, _LANES)
                    sv[b][i, sl] = sv[b][i, sl] * xv[b][i, sl]
                return 0
            lax.fori_loop(0, _CHUNK, erow, 0)

            pltpu.sync_copy(sv[b], acc.at[dstv[b]], add=True)

            cj1 = jnp.minimum(cj + 1, nchunk - 1)
            wait_loads(cj1, nb)
            gather_descr(nb).start()
            cj2 = jnp.minimum(cj + 2, nchunk - 1)
            issue_loads(cj2, b)

        issue_loads(0, 0)
        wait_loads(0, 0)
        gather_descr(0).start()
        issue_loads(1, 1)
        def pair(j, _):
            process(2 * j, 0, 1)
            process(2 * j + 1, 1, 0)
            return 0
        lax.fori_loop(0, npairs, pair, 0)
        gather_descr(0).wait()
        wait_loads(nchunk - 1, 1)
        plsc.subcore_barrier()

        pltpu.sync_copy(
            acc.at[pl.ds(s * rpt, rpt), :],
            out_hbm.at[pl.ds(c * npad + s * rpt, rpt), :])

    return sc_kernel


def kernel(x, edge_index, edge_attr, node_attr, W_in, b_in, W_out, b_out):
    n, d = x.shape
    na = node_attr.shape[1]
    e = edge_index.shape[1]
    ea = edge_attr.shape[1]
    ic = d + na
    oc = W_out.shape[0]
    icp = ((ic + _LANES - 1) // _LANES) * _LANES
    npad = ((n + _NS * 16 - 1) // (_NS * 16)) * (_NS * 16)

    f32 = jnp.float32
    src = edge_index[0].astype(jnp.int32)
    dst = edge_index[1].astype(jnp.int32)

    xc = jnp.concatenate([x.astype(f32), node_attr.astype(f32)], axis=1)
    xcp = jnp.pad(xc, ((0, 0), (0, icp - ic)))

    w_h = [jnp.pad(W_in[h::2, :], ((0, icp - ic), (0, 0))) for h in range(2)]
    w_cat = jnp.concatenate(w_h, axis=0).T.astype(f32)
    b_h = [jnp.pad(b_in[h::2], (0, icp - ic)) for h in range(2)]
    b_cat = jnp.concatenate(b_h, axis=0)[None, :].astype(f32)

    tile_e = 640
    scal = pl.pallas_call(
        functools.partial(_tc_scaling_body, icp=icp),
        grid=(e // tile_e,),
        in_specs=[
            pl.BlockSpec((tile_e, ea), lambda i: (i, 0)),
            pl.BlockSpec((ea, 2 * icp), lambda i: (0, 0)),
            pl.BlockSpec((1, 2 * icp), lambda i: (0, 0)),
        ],
        out_specs=pl.BlockSpec((2, tile_e, icp), lambda i: (0, i, 0)),
        out_shape=jax.ShapeDtypeStruct((2, e, icp), f32),
    )(edge_attr.astype(f32), w_cat, b_cat)
    scal_flat = scal.reshape(2 * e, icp)

    sc_fn = _sc_gather_scale_scatter(e, icp, npad)
    aggr = sc_fn(xcp, src, dst, scal_flat)
    a0 = aggr[:n]
    a1 = aggr[npad:npad + n]

    w0o = jnp.pad(W_out[:, 0::2], ((0, 0), (0, icp - ic))).T.astype(f32)
    w1o = jnp.pad(W_out[:, 1::2], ((0, 0), (0, icp - ic))).T.astype(f32)
    b_o = b_out[None, :].astype(f32)

    tile_n = 1000
    out = pl.pallas_call(
        _tc_out_body,
        grid=(n // tile_n,),
        in_specs=[
            pl.BlockSpec((tile_n, icp), lambda i: (i, 0)),
            pl.BlockSpec((tile_n, icp), lambda i: (i, 0)),
            pl.BlockSpec((icp, oc), lambda i: (0, 0)),
            pl.BlockSpec((icp, oc), lambda i: (0, 0)),
            pl.BlockSpec((1, oc), lambda i: (0, 0)),
        ],
        out_specs=pl.BlockSpec((tile_n, oc), lambda i: (i, 0)),
        out_shape=jax.ShapeDtypeStruct((n, oc), f32),
    )(a0, a1, w0o, w1o, b_o)
    return out


# speedup vs baseline: 1.7665x; 1.0259x over previous
import functools

import jax
import jax.numpy as jnp
from jax import lax
from jax.experimental import pallas as pl
from jax.experimental.pallas import tpu as pltpu
from jax.experimental.pallas import tpu_sc as plsc

_LANES = 16
_NC = 2
_NS = 16
_CHUNK = 40


def _tc_scaling_body(ea_ref, w_ref, b_ref, out_ref, *, icp):
    s = jnp.dot(ea_ref[...], w_ref[...], preferred_element_type=jnp.float32)
    s = jnp.maximum(s + b_ref[...], 0.0)
    out_ref[0] = s[:, :icp]
    out_ref[1] = s[:, icp:]


def _tc_out_body(a0_ref, a1_ref, w0_ref, w1_ref, b_ref, out_ref):
    acc = jnp.dot(a0_ref[...], w0_ref[...], preferred_element_type=jnp.float32)
    acc = acc + jnp.dot(a1_ref[...], w1_ref[...], preferred_element_type=jnp.float32)
    out_ref[...] = jnp.tanh(acc + b_ref[...])


def _sc_gather_scale_scatter(e, icp, npad):
    ept = e // _NS
    nchunk = ept // _CHUNK
    rpt = npad // _NS
    nvec = icp // _LANES
    zrows = 16

    npairs = nchunk // 2

    mesh = plsc.VectorSubcoreMesh(
        core_axis_name="c", subcore_axis_name="s",
        num_cores=_NC, num_subcores=_NS)

    @functools.partial(
        pl.kernel,
        out_type=jax.ShapeDtypeStruct((_NC * npad, icp), jnp.float32),
        mesh=mesh,
        scratch_types=[
            pltpu.VMEM((_CHUNK,), jnp.int32),
            pltpu.VMEM((_CHUNK,), jnp.int32),
            pltpu.VMEM((_CHUNK,), jnp.int32),
            pltpu.VMEM((_CHUNK,), jnp.int32),
            pltpu.VMEM((_CHUNK, icp), jnp.float32),
            pltpu.VMEM((_CHUNK, icp), jnp.float32),
            pltpu.VMEM((_CHUNK, icp), jnp.float32),
            pltpu.VMEM((_CHUNK, icp), jnp.float32),
            pltpu.VMEM_SHARED((npad, icp), jnp.float32),
            pltpu.VMEM((zrows, icp), jnp.float32),
            pltpu.SemaphoreType.DMA,
            pltpu.SemaphoreType.DMA,
            pltpu.SemaphoreType.DMA,
            pltpu.SemaphoreType.DMA,
        ],
        compiler_params=pltpu.CompilerParams(use_tc_tiling_on_sc=False),
    )
    def sc_kernel(xc_hbm, src_hbm, dst_hbm, s_hbm, out_hbm,
                  srcv0, srcv1, dstv0, dstv1, sv0, sv1, xv0, xv1,
                  acc, zbuf, ls0, ls1, gs0, gs1):
        c = lax.axis_index("c")
        s = lax.axis_index("s")
        srcv = (srcv0, srcv1)
        dstv = (dstv0, dstv1)
        sv = (sv0, sv1)
        xv = (xv0, xv1)
        ls = (ls0, ls1)
        gs = (gs0, gs1)

        def zrow(r, _):
            for k in range(nvec):
                zbuf[r, pl.ds(k * _LANES, _LANES)] = jnp.zeros(
                    (_LANES,), jnp.float32)
            return 0
        lax.fori_loop(0, zrows, zrow, 0)

        def zcopy(j, _):
            pltpu.sync_copy(
                zbuf, acc.at[pl.ds(s * rpt + j * zrows, zrows), :])
            return 0
        lax.fori_loop(0, rpt // zrows, zcopy, 0)
        plsc.subcore_barrier()

        def loads_descr(cj, b):
            base = s * ept + cj * _CHUNK
            return (
                pltpu.make_async_copy(
                    src_hbm.at[pl.ds(base, _CHUNK)], srcv[b], ls[b]),
                pltpu.make_async_copy(
                    dst_hbm.at[pl.ds(base, _CHUNK)], dstv[b], ls[b]),
                pltpu.make_async_copy(
                    s_hbm.at[pl.ds(c * e + base, _CHUNK), :], sv[b], ls[b]),
            )

        def issue_loads(cj, b):
            for d in loads_descr(cj, b):
                d.start()

        def wait_loads(cj, b):
            for d in loads_descr(cj, b):
                d.wait()

        def gather_descr(b):
            return pltpu.make_async_copy(xc_hbm.at[srcv[b]], xv[b], gs[b])

        def process(cj, b, nb):
            cj1 = jnp.minimum(cj + 1, nchunk - 1)
            wait_loads(cj1, nb)
            gather_descr(nb).start()
            gather_descr(b).wait()

            def erow(i, _):
                for k in range(nvec):
                    sl = pl.ds(k * _LANES, _LANES)
                    sv[b][i, sl] = sv[b][i, sl] * xv[b][i, sl]
                return 0
            lax.fori_loop(0, _CHUNK, erow, 0)

            pltpu.sync_copy(sv[b], acc.at[dstv[b]], add=True)

            cj2 = jnp.minimum(cj + 2, nchunk - 1)
            issue_loads(cj2, b)

        issue_loads(0, 0)
        wait_loads(0, 0)
        gather_descr(0).start()
        issue_loads(1, 1)
        def pair(j, _):
            process(2 * j, 0, 1)
            process(2 * j + 1, 1, 0)
            return 0
        lax.fori_loop(0, npairs, pair, 0)
        gather_descr(0).wait()
        wait_loads(nchunk - 1, 1)
        plsc.subcore_barrier()

        pltpu.sync_copy(
            acc.at[pl.ds(s * rpt, rpt), :],
            out_hbm.at[pl.ds(c * npad + s * rpt, rpt), :])

    return sc_kernel


def kernel(x, edge_index, edge_attr, node_attr, W_in, b_in, W_out, b_out):
    n, d = x.shape
    na = node_attr.shape[1]
    e = edge_index.shape[1]
    ea = edge_attr.shape[1]
    ic = d + na
    oc = W_out.shape[0]
    icp = ((ic + _LANES - 1) // _LANES) * _LANES
    npad = ((n + _NS * 16 - 1) // (_NS * 16)) * (_NS * 16)

    f32 = jnp.float32
    src = edge_index[0].astype(jnp.int32)
    dst = edge_index[1].astype(jnp.int32)

    xc = jnp.concatenate([x.astype(f32), node_attr.astype(f32)], axis=1)
    xcp = jnp.pad(xc, ((0, 0), (0, icp - ic)))

    w_h = [jnp.pad(W_in[h::2, :], ((0, icp - ic), (0, 0))) for h in range(2)]
    w_cat = jnp.concatenate(w_h, axis=0).T.astype(f32)
    b_h = [jnp.pad(b_in[h::2], (0, icp - ic)) for h in range(2)]
    b_cat = jnp.concatenate(b_h, axis=0)[None, :].astype(f32)

    tile_e = 640
    scal = pl.pallas_call(
        functools.partial(_tc_scaling_body, icp=icp),
        grid=(e // tile_e,),
        in_specs=[
            pl.BlockSpec((tile_e, ea), lambda i: (i, 0)),
            pl.BlockSpec((ea, 2 * icp), lambda i: (0, 0)),
            pl.BlockSpec((1, 2 * icp), lambda i: (0, 0)),
        ],
        out_specs=pl.BlockSpec((2, tile_e, icp), lambda i: (0, i, 0)),
        out_shape=jax.ShapeDtypeStruct((2, e, icp), f32),
    )(edge_attr.astype(f32), w_cat, b_cat)
    scal_flat = scal.reshape(2 * e, icp)

    sc_fn = _sc_gather_scale_scatter(e, icp, npad)
    aggr = sc_fn(xcp, src, dst, scal_flat)
    a0 = aggr[:n]
    a1 = aggr[npad:npad + n]

    w0o = jnp.pad(W_out[:, 0::2], ((0, 0), (0, icp - ic))).T.astype(f32)
    w1o = jnp.pad(W_out[:, 1::2], ((0, 0), (0, icp - ic))).T.astype(f32)
    b_o = b_out[None, :].astype(f32)

    tile_n = 1000
    out = pl.pallas_call(
        _tc_out_body,
        grid=(n // tile_n,),
        in_specs=[
            pl.BlockSpec((tile_n, icp), lambda i: (i, 0)),
            pl.BlockSpec((tile_n, icp), lambda i: (i, 0)),
            pl.BlockSpec((icp, oc), lambda i: (0, 0)),
            pl.BlockSpec((icp, oc), lambda i: (0, 0)),
            pl.BlockSpec((1, oc), lambda i: (0, 0)),
        ],
        out_specs=pl.BlockSpec((tile_n, oc), lambda i: (i, 0)),
        out_shape=jax.ShapeDtypeStruct((n, oc), f32),
    )(a0, a1, w0o, w1o, b_o)
    return out
